# baseline (speedup 1.0000x reference)
import jax
import jax.numpy as jnp
from jax.experimental import pallas as pl
from jax.experimental.pallas import tpu as pltpu

N = 4096
F_IN = 128
F_OUT = 64
NHEADS = 4
ALPHA = 0.2

BR = 256
BC = 4096
NI = N // BR
HW = 128


def _aug(wh, fout):
    br = wh.shape[0]
    nheads = wh.shape[1] // fout
    ones = jnp.ones((br, 1), jnp.float32)
    zpad = jnp.zeros((br, HW - fout - 1), jnp.float32)
    parts = []
    for k in range(nheads):
        parts += [wh[:, k * fout:(k + 1) * fout], ones, zpad]
    return jnp.concatenate(parts, axis=1).astype(jnp.bfloat16)


def _row_factors(s1c, s2m):
    t = s1c + s2m
    mrow = jnp.maximum(jnp.where(t >= 0.0, t, ALPHA * t), 0.0)
    row = jnp.concatenate(
        [jnp.exp(t - mrow),
         jnp.exp(ALPHA * t - mrow),
         jnp.exp(-mrow),
         -s1c], axis=1)
    return row.astype(jnp.bfloat16)


def _head_probs(mask, s2r, s2m, row, k):
    nh = row.shape[1] // 4
    e1 = row[:, k:k + 1]
    f1 = row[:, nh + k:nh + k + 1]
    g = row[:, 2 * nh + k:2 * nh + k + 1]
    ns1 = row[:, 3 * nh + k:3 * nh + k + 1]
    e2 = jnp.exp(s2r - s2m).astype(jnp.bfloat16)
    f2 = jnp.exp(ALPHA * (s2r - s2m)).astype(jnp.bfloat16)
    c = s2r.astype(jnp.bfloat16) >= ns1
    u = jnp.where(c, e2, f2)
    v = jnp.where(c, e1, f1)
    return jnp.where(mask, u * v, g)


def _elu(x):
    return jnp.where(x > 0.0, x, jnp.exp(x) - 1.0)


def _fused_kernel(h_ref, adj_ref, w_ref, a1_ref, a2_ref,
                  wout_ref, a1o_ref, a2o_ref, out_ref,
                  wh_scr, s1_scr, s2t_scr, s2max_scr,
                  mask_scr, who_scr, s1o_scr, s2to_scr, s2mo_scr):
    t = pl.program_id(0)
    i = pl.program_id(1)
    ni = pl.num_programs(1)
    fout = F_OUT
    rows = pl.ds(i * BR, BR)

    @pl.when(t == 0)
    def _prep():
        wh = jnp.dot(h_ref[rows, :], w_ref[:],
                     preferred_element_type=jnp.float32)
        s1 = jnp.dot(wh, a1_ref[:], preferred_element_type=jnp.float32)
        s2 = jnp.dot(wh, a2_ref[:], preferred_element_type=jnp.float32)
        wh_scr[rows, :] = _aug(wh, fout)
        s1_scr[rows, :] = s1
        s2t_scr[:, rows] = jnp.concatenate(
            [s2.T, jnp.zeros((8 - NHEADS, BR), jnp.float32)], axis=0)
        bmax = jnp.max(s2, axis=0, keepdims=True)

        @pl.when(i == 0)
        def _():
            s2max_scr[:] = jnp.full_like(s2max_scr, -jnp.inf)

        s2max_scr[:] = jnp.maximum(s2max_scr[:], bmax)

    @pl.when(t == 1)
    def _heads():
        row = _row_factors(s1_scr[rows, :], s2max_scr[0:1, :])
        mask = adj_ref[:].astype(jnp.bfloat16) > 0.0
        mask_scr[rows, :] = mask.astype(jnp.int8)

        hc_parts = []
        for k in range(NHEADS):
            p = _head_probs(mask, s2t_scr[k:k + 1, :],
                            s2max_scr[0:1, k:k + 1], row, k)
            acc = jnp.dot(p, wh_scr[:, k * HW:(k + 1) * HW],
                          preferred_element_type=jnp.float32)
            hp = acc[:, :fout] / acc[:, fout:fout + 1]
            hc_parts.append(_elu(hp))
        hcb = jnp.concatenate(hc_parts, axis=1)

        who = jnp.dot(hcb, wout_ref[:], preferred_element_type=jnp.float32)
        s1o = jnp.dot(who, a1o_ref[:], preferred_element_type=jnp.float32)
        s2o = jnp.dot(who, a2o_ref[:], preferred_element_type=jnp.float32)

        who_scr[rows, :] = _aug(who, fout)
        s1o_scr[rows, :] = s1o
        s2to_scr[:, rows] = jnp.concatenate(
            [s2o.T, jnp.zeros((7, BR), jnp.float32)], axis=0)

        @pl.when(i == 0)
        def _():
            s2mo_scr[:] = jnp.full_like(s2mo_scr, -jnp.inf)

        s2mo_scr[:] = jnp.maximum(s2mo_scr[:], jnp.max(s2o))

    @pl.when(t == 2)
    def _outhead():
        s2m = s2mo_scr[0:1, 0:1]
        row = _row_factors(s1o_scr[rows, :], s2m)
        mask = mask_scr[rows, :].astype(jnp.bfloat16) > 0.0
        p = _head_probs(mask, s2to_scr[0:1, :], s2m, row, 0)
        acc = jnp.dot(p, who_scr[:], preferred_element_type=jnp.float32)
        hp = acc[:, :fout] / acc[:, fout:fout + 1]
        y = _elu(hp)
        mx = jnp.max(y, axis=1, keepdims=True)
        lse = jnp.log(jnp.sum(jnp.exp(y - mx), axis=1, keepdims=True))
        out_ref[:] = y - mx - lse


def kernel(x, adj, W0, W1, W2, W3, a0, a1, a2, a3, Wout, aout):
    h = x.reshape(N, F_IN)
    adjm = adj.reshape(N, N)

    wcat = jnp.concatenate([W0, W1, W2, W3], axis=1)
    a_list = [a0, a1, a2, a3]
    eye = jnp.eye(NHEADS, dtype=jnp.float32)
    a1cat = jnp.concatenate(
        [a_list[k][:F_OUT] * eye[k] for k in range(NHEADS)], axis=0)
    a2cat = jnp.concatenate(
        [a_list[k][F_OUT:] * eye[k] for k in range(NHEADS)], axis=0)

    out = pl.pallas_call(
        _fused_kernel,
        grid=(3, NI),
        in_specs=[
            pl.BlockSpec((N, F_IN), lambda t, i: (0, 0)),
            pl.BlockSpec(
                (BR, BC),
                lambda t, i: (jnp.where(t == 1, i,
                                        jnp.where(t == 0, 0, NI - 1)), 0)),
            pl.BlockSpec((F_IN, NHEADS * F_OUT), lambda t, i: (0, 0)),
            pl.BlockSpec((NHEADS * F_OUT, NHEADS), lambda t, i: (0, 0)),
            pl.BlockSpec((NHEADS * F_OUT, NHEADS), lambda t, i: (0, 0)),
            pl.BlockSpec((NHEADS * F_OUT, F_OUT), lambda t, i: (0, 0)),
            pl.BlockSpec((F_OUT, 1), lambda t, i: (0, 0)),
            pl.BlockSpec((F_OUT, 1), lambda t, i: (0, 0)),
        ],
        out_specs=pl.BlockSpec((BR, F_OUT),
                               lambda t, i: (jnp.where(t == 2, i, 0), 0)),
        out_shape=jax.ShapeDtypeStruct((N, F_OUT), jnp.float32),
        scratch_shapes=[
            pltpu.VMEM((N, NHEADS * HW), jnp.bfloat16),
            pltpu.VMEM((N, NHEADS), jnp.float32),
            pltpu.VMEM((8, N), jnp.float32),
            pltpu.VMEM((1, NHEADS), jnp.float32),
            pltpu.VMEM((N, N), jnp.int8),
            pltpu.VMEM((N, HW), jnp.bfloat16),
            pltpu.VMEM((N, 1), jnp.float32),
            pltpu.VMEM((8, N), jnp.float32),
            pltpu.VMEM((1, 1), jnp.float32),
        ],
        compiler_params=pltpu.CompilerParams(
            dimension_semantics=("arbitrary", "arbitrary")),
    )(h, adjm, wcat, a1cat, a2cat, Wout, aout[:F_OUT], aout[F_OUT:])
    return out


# speedup vs baseline: 1.1756x; 1.1756x over previous
import jax
import jax.numpy as jnp
from jax.experimental import pallas as pl
from jax.experimental.pallas import tpu as pltpu

N = 4096
F_IN = 128
F_OUT = 64
NHEADS = 4
ALPHA = 0.2

BR = 512
BC = 4096
NI = N // BR
HW = 128


def _aug(wh, fout):
    br = wh.shape[0]
    nheads = wh.shape[1] // fout
    ones = jnp.ones((br, 1), jnp.float32)
    zpad = jnp.zeros((br, HW - fout - 1), jnp.float32)
    parts = []
    for k in range(nheads):
        parts += [wh[:, k * fout:(k + 1) * fout], ones, zpad]
    return jnp.concatenate(parts, axis=1).astype(jnp.bfloat16)


def _row_factors(s1c, s2m):
    t = s1c + s2m
    mrow = jnp.maximum(jnp.where(t >= 0.0, t, ALPHA * t), 0.0)
    row = jnp.concatenate(
        [jnp.exp(t - mrow),
         jnp.exp(ALPHA * t - mrow),
         jnp.exp(-mrow),
         -s1c], axis=1)
    return row.astype(jnp.bfloat16)


def _head_probs(mask, s2r, s2m, row, k):
    nh = row.shape[1] // 4
    e1 = row[:, k:k + 1]
    f1 = row[:, nh + k:nh + k + 1]
    g = row[:, 2 * nh + k:2 * nh + k + 1]
    ns1 = row[:, 3 * nh + k:3 * nh + k + 1]
    e2 = jnp.exp(s2r - s2m).astype(jnp.bfloat16)
    f2 = jnp.exp(ALPHA * (s2r - s2m)).astype(jnp.bfloat16)
    c = s2r.astype(jnp.bfloat16) >= ns1
    u = jnp.where(c, e2, f2)
    v = jnp.where(c, e1, f1)
    return jnp.where(mask, u * v, g)


def _elu(x):
    return jnp.where(x > 0.0, x, jnp.exp(x) - 1.0)


def _fused_kernel(h_ref, adj_ref, w_ref, a1_ref, a2_ref,
                  wout_ref, a1o_ref, a2o_ref, out_ref,
                  wh_scr, s1_scr, s2t_scr, s2max_scr,
                  mask_scr, who_scr, s1o_scr, s2to_scr, s2mo_scr):
    t = pl.program_id(0)
    i = pl.program_id(1)
    ni = pl.num_programs(1)
    fout = F_OUT
    rows = pl.ds(i * BR, BR)

    @pl.when(t == 0)
    def _prep():
        wh = jnp.dot(h_ref[rows, :], w_ref[:],
                     preferred_element_type=jnp.float32)
        s1 = jnp.dot(wh, a1_ref[:], preferred_element_type=jnp.float32)
        s2 = jnp.dot(wh, a2_ref[:], preferred_element_type=jnp.float32)
        wh_scr[rows, :] = _aug(wh, fout)
        s1_scr[rows, :] = s1
        s2t_scr[:, rows] = jnp.concatenate(
            [s2.T, jnp.zeros((8 - NHEADS, BR), jnp.float32)], axis=0)
        bmax = jnp.max(s2, axis=0, keepdims=True)

        @pl.when(i == 0)
        def _():
            s2max_scr[:] = jnp.full_like(s2max_scr, -jnp.inf)

        s2max_scr[:] = jnp.maximum(s2max_scr[:], bmax)

    @pl.when(t == 1)
    def _heads():
        row = _row_factors(s1_scr[rows, :], s2max_scr[0:1, :])
        mask = adj_ref[:].astype(jnp.bfloat16) > 0.0
        mask_scr[rows, :] = mask.astype(jnp.int8)

        hc_parts = []
        for k in range(NHEADS):
            p = _head_probs(mask, s2t_scr[k:k + 1, :],
                            s2max_scr[0:1, k:k + 1], row, k)
            acc = jnp.dot(p, wh_scr[:, k * HW:(k + 1) * HW],
                          preferred_element_type=jnp.float32)
            hp = acc[:, :fout] / acc[:, fout:fout + 1]
            hc_parts.append(_elu(hp))
        hcb = jnp.concatenate(hc_parts, axis=1)

        who = jnp.dot(hcb, wout_ref[:], preferred_element_type=jnp.float32)
        s1o = jnp.dot(who, a1o_ref[:], preferred_element_type=jnp.float32)
        s2o = jnp.dot(who, a2o_ref[:], preferred_element_type=jnp.float32)

        who_scr[rows, :] = _aug(who, fout)
        s1o_scr[rows, :] = s1o
        s2to_scr[:, rows] = jnp.concatenate(
            [s2o.T, jnp.zeros((7, BR), jnp.float32)], axis=0)

        @pl.when(i == 0)
        def _():
            s2mo_scr[:] = jnp.full_like(s2mo_scr, -jnp.inf)

        s2mo_scr[:] = jnp.maximum(s2mo_scr[:], jnp.max(s2o))

    @pl.when(t == 2)
    def _outhead():
        s2m = s2mo_scr[0:1, 0:1]
        row = _row_factors(s1o_scr[rows, :], s2m)
        mask = mask_scr[rows, :].astype(jnp.bfloat16) > 0.0
        p = _head_probs(mask, s2to_scr[0:1, :], s2m, row, 0)
        acc = jnp.dot(p, who_scr[:], preferred_element_type=jnp.float32)
        hp = acc[:, :fout] / acc[:, fout:fout + 1]
        y = _elu(hp)
        mx = jnp.max(y, axis=1, keepdims=True)
        lse = jnp.log(jnp.sum(jnp.exp(y - mx), axis=1, keepdims=True))
        out_ref[:] = y - mx - lse


def kernel(x, adj, W0, W1, W2, W3, a0, a1, a2, a3, Wout, aout):
    h = x.reshape(N, F_IN)
    adjm = adj.reshape(N, N)

    wcat = jnp.concatenate([W0, W1, W2, W3], axis=1)
    a_list = [a0, a1, a2, a3]
    eye = jnp.eye(NHEADS, dtype=jnp.float32)
    a1cat = jnp.concatenate(
        [a_list[k][:F_OUT] * eye[k] for k in range(NHEADS)], axis=0)
    a2cat = jnp.concatenate(
        [a_list[k][F_OUT:] * eye[k] for k in range(NHEADS)], axis=0)

    out = pl.pallas_call(
        _fused_kernel,
        grid=(3, NI),
        in_specs=[
            pl.BlockSpec((N, F_IN), lambda t, i: (0, 0)),
            pl.BlockSpec(
                (BR, BC),
                lambda t, i: (jnp.where(t == 1, i,
                                        jnp.where(t == 0, 0, NI - 1)), 0)),
            pl.BlockSpec((F_IN, NHEADS * F_OUT), lambda t, i: (0, 0)),
            pl.BlockSpec((NHEADS * F_OUT, NHEADS), lambda t, i: (0, 0)),
            pl.BlockSpec((NHEADS * F_OUT, NHEADS), lambda t, i: (0, 0)),
            pl.BlockSpec((NHEADS * F_OUT, F_OUT), lambda t, i: (0, 0)),
            pl.BlockSpec((F_OUT, 1), lambda t, i: (0, 0)),
            pl.BlockSpec((F_OUT, 1), lambda t, i: (0, 0)),
        ],
        out_specs=pl.BlockSpec((BR, F_OUT),
                               lambda t, i: (jnp.where(t == 2, i, 0), 0)),
        out_shape=jax.ShapeDtypeStruct((N, F_OUT), jnp.float32),
        scratch_shapes=[
            pltpu.VMEM((N, NHEADS * HW), jnp.bfloat16),
            pltpu.VMEM((N, NHEADS), jnp.float32),
            pltpu.VMEM((8, N), jnp.float32),
            pltpu.VMEM((1, NHEADS), jnp.float32),
            pltpu.VMEM((N, N), jnp.int8),
            pltpu.VMEM((N, HW), jnp.bfloat16),
            pltpu.VMEM((N, 1), jnp.float32),
            pltpu.VMEM((8, N), jnp.float32),
            pltpu.VMEM((1, 1), jnp.float32),
        ],
        compiler_params=pltpu.CompilerParams(
            dimension_semantics=("arbitrary", "arbitrary")),
    )(h, adjm, wcat, a1cat, a2cat, Wout, aout[:F_OUT], aout[F_OUT:])
    return out
